# baseline (speedup 1.0000x reference)
import jax
import jax.numpy as jnp
from jax import lax
from jax.experimental import pallas as pl
from jax.experimental.pallas import tpu as pltpu
from jax.experimental.pallas import tpu_sc as plsc

IOU_THRES = 0.1
N_BOXES = 20000
NT = 16
SHARD = 1280
NPAD = NT * SHARD
NV = SHARD // 16
BIG = jnp.int32(2**31 - 1)


def _lane():
    return lax.broadcasted_iota(jnp.int32, (16,), 0)


def _splat_f(x):
    return jnp.full((16,), x, jnp.float32)


def _splat_i(x):
    return jnp.full((16,), x, jnp.int32)


def _sc_body(cx_hbm, cy_hbm, w_hbm, h_hbm, s_hbm, out_hbm,
             sx1, sy1, sx2, sy2, sarea, sscore, sglin, souts,
             pubrow_v, pub_l, pub_sh):
    sid = lax.axis_index("s")
    base = sid * SHARD
    lane = _lane()
    neg_inf = jnp.float32(-jnp.inf)
    ninf_v = _splat_f(neg_inf)

    pltpu.sync_copy(cx_hbm.at[pl.ds(base, SHARD)], sx1)
    pltpu.sync_copy(w_hbm.at[pl.ds(base, SHARD)], sx2)
    pltpu.sync_copy(cy_hbm.at[pl.ds(base, SHARD)], sy1)
    pltpu.sync_copy(h_hbm.at[pl.ds(base, SHARD)], sy2)
    pltpu.sync_copy(s_hbm.at[pl.ds(base, SHARD)], sscore)

    cnt0 = jnp.minimum(jnp.maximum(N_BOXES - base, 0), SHARD)

    def init_body(k, carry):
        bm, bgl, bpos = carry
        sl = pl.ds(k * 16, 16)
        va = sx1[sl]
        vb = sx2[sl]
        x1 = va - vb / 2.0
        x2 = va + vb / 2.0
        sx1[sl] = x1
        sx2[sl] = x2
        vc = sy1[sl]
        vd = sy2[sl]
        y1 = vc - vd / 2.0
        y2 = vc + vd / 2.0
        sy1[sl] = y1
        sy2[sl] = y2
        sarea[sl] = (x2 - x1) * (y2 - y1)
        pos = k * 16 + lane
        gl = base + pos
        sglin[sl] = gl
        souts[sl] = jnp.zeros((16,), jnp.float32)
        sc = sscore[sl]
        scm = jnp.where(pos < cnt0, sc, ninf_v)
        better = (scm > bm) | ((scm == bm) & (scm > ninf_v) & (gl < bgl))
        return (jnp.where(better, scm, bm), jnp.where(better, gl, bgl),
                jnp.where(better, pos, bpos))

    bm, bgl, bpos = lax.fori_loop(
        0, NV, init_body, (ninf_v, _splat_i(BIG), _splat_i(0)))

    def publish_and_read(bm, bgl, bpos):
        m = jnp.max(bm)
        lgl = jnp.min(jnp.where((bm == m) & (bm > ninf_v), bgl, BIG))
        lpos = jnp.min(jnp.where((bm == m) & (bgl == lgl), bpos, BIG))
        posv = _splat_i(lpos)
        wx1 = plsc.load_gather(sx1, [posv])
        wy1 = plsc.load_gather(sy1, [posv])
        wx2 = plsc.load_gather(sx2, [posv])
        wy2 = plsc.load_gather(sy2, [posv])
        war = plsc.load_gather(sarea, [posv])
        row = _splat_f(m)
        row = jnp.where(lane == 1, plsc.bitcast(_splat_i(lgl), jnp.float32),
                        row)
        row = jnp.where(lane == 2, wx1, row)
        row = jnp.where(lane == 3, wy1, row)
        row = jnp.where(lane == 4, wx2, row)
        row = jnp.where(lane == 5, wy2, row)
        row = jnp.where(lane == 6, war, row)
        pubrow_v[...] = row
        pltpu.sync_copy(pubrow_v, pub_sh.at[sid])
        plsc.subcore_barrier()
        pltpu.sync_copy(pub_sh, pub_l)
        plsc.subcore_barrier()
        mvec = plsc.load_gather(pub_l, [lane, _splat_i(0)])
        glvec = plsc.bitcast(plsc.load_gather(pub_l, [lane, _splat_i(1)]),
                             jnp.int32)
        gmax = jnp.max(mvec)
        gwin = jnp.min(jnp.where(mvec == gmax, glvec, BIG))
        wrow = _splat_i(jnp.where(gmax > neg_inf, gwin // SHARD, 0))
        bx1 = plsc.load_gather(pub_l, [wrow, _splat_i(2)])
        by1 = plsc.load_gather(pub_l, [wrow, _splat_i(3)])
        bx2 = plsc.load_gather(pub_l, [wrow, _splat_i(4)])
        by2 = plsc.load_gather(pub_l, [wrow, _splat_i(5)])
        bar = plsc.load_gather(pub_l, [wrow, _splat_i(6)])
        return gmax, gwin, bx1, by1, bx2, by2, bar

    gmax, gwin, bx1, by1, bx2, by2, bar = publish_and_read(bm, bgl, bpos)
    state0 = (jnp.int32(1), gmax, gwin, bx1, by1, bx2, by2, bar, cnt0)

    def cond(state):
        return state[1] > neg_inf

    def loop_body(state):
        it, gmax, gwin, bx1, by1, bx2, by2, bar, count = state
        nv = (count + 15) // 16

        def step(k, carry):
            woff, bm, bgl, bpos = carry
            sl = pl.ds(k * 16, 16)
            x1 = sx1[sl]
            y1 = sy1[sl]
            x2 = sx2[sl]
            y2 = sy2[sl]
            ar = sarea[sl]
            sc = sscore[sl]
            gl = sglin[sl]
            ix1 = jnp.maximum(bx1, x1)
            iy1 = jnp.maximum(by1, y1)
            ix2 = jnp.minimum(bx2, x2)
            iy2 = jnp.minimum(by2, y2)
            zero = jnp.float32(0.0)
            inter = jnp.maximum(ix2 - ix1, zero) * jnp.maximum(iy2 - iy1,
                                                               zero)
            iou = inter / (bar + ar - inter + 1e-9)
            inb = (k * 16 + lane) < count
            alive = inb & jnp.logical_not(iou > IOU_THRES) & (gl != gwin)
            cs = plsc.cumsum(alive.astype(jnp.int32))
            idx = woff + cs - 1
            plsc.store_scatter(sx1, [idx], x1, mask=alive)
            plsc.store_scatter(sy1, [idx], y1, mask=alive)
            plsc.store_scatter(sx2, [idx], x2, mask=alive)
            plsc.store_scatter(sy2, [idx], y2, mask=alive)
            plsc.store_scatter(sarea, [idx], ar, mask=alive)
            plsc.store_scatter(sscore, [idx], sc, mask=alive)
            plsc.store_scatter(sglin, [idx], gl, mask=alive)
            scm = jnp.where(alive, sc, ninf_v)
            better = (scm > bm) | ((scm == bm) & (scm > ninf_v) & (gl < bgl))
            bm = jnp.where(better, scm, bm)
            bgl = jnp.where(better, gl, bgl)
            bpos = jnp.where(better, idx, bpos)
            return woff + jnp.max(cs), bm, bgl, bpos

        woff, bm, bgl, bpos = lax.fori_loop(
            0, nv, step, (jnp.int32(0), ninf_v, _splat_i(BIG), _splat_i(0)))

        mine = (gwin >= base) & (gwin < base + SHARD)

        @pl.when(mine)
        def _():
            plsc.store_scatter(souts, [_splat_i(gwin - base)],
                               _splat_f(gmax), mask=lane == 0)

        res = publish_and_read(bm, bgl, bpos)
        return (it + 1,) + res + (woff,)

    lax.while_loop(cond, loop_body, state0)
    pltpu.sync_copy(souts, out_hbm.at[pl.ds(base, SHARD)])


def kernel(boxes, scores):
    n = scores.shape[0]
    pad = NPAD - n
    b = jnp.pad(boxes, ((0, pad), (0, 0)))
    s = jnp.pad(scores, (0, pad))
    mesh = plsc.VectorSubcoreMesh(core_axis_name="c", subcore_axis_name="s",
                                  num_cores=1, num_subcores=NT)
    f32 = jnp.float32
    out = pl.kernel(
        _sc_body,
        out_type=jax.ShapeDtypeStruct((NPAD,), f32),
        mesh=mesh,
        scratch_types=[
            pltpu.VMEM((SHARD,), f32),
            pltpu.VMEM((SHARD,), f32),
            pltpu.VMEM((SHARD,), f32),
            pltpu.VMEM((SHARD,), f32),
            pltpu.VMEM((SHARD,), f32),
            pltpu.VMEM((SHARD,), f32),
            pltpu.VMEM((SHARD,), jnp.int32),
            pltpu.VMEM((SHARD,), f32),
            pltpu.VMEM((16,), f32),
            pltpu.VMEM((16, 16), f32),
            pltpu.VMEM_SHARED((16, 16), f32),
        ],
        compiler_params=pltpu.CompilerParams(needs_layout_passes=False),
    )(b[:, 0], b[:, 1], b[:, 2], b[:, 3], s)
    return out[:n]


# speedup vs baseline: 1.5728x; 1.5728x over previous
import jax
import jax.numpy as jnp
import numpy as np
from jax import lax
from jax.experimental import pallas as pl
from jax.experimental.pallas import tpu as pltpu
from jax.experimental.pallas import tpu_sc as plsc

IOU_THRES = 0.1
N = 20000
NT = 16
SHARD = 1280
NPAD = NT * SHARD
NV = SHARD // 16
BIG = jnp.int32(2**31 - 1)


def _lane():
    return lax.broadcasted_iota(jnp.int32, (16,), 0)


def _splat_f(x):
    return jnp.full((16,), x, jnp.float32)


def _splat_i(x):
    return jnp.full((16,), x, jnp.int32)


def _top2_update(carry, v, gl, pos, ninf_v):
    m1, g1, p1, m2, g2, p2 = carry
    b1 = (v > m1) | ((v == m1) & (v > ninf_v) & (gl < g1))
    b2 = jnp.logical_not(b1) & ((v > m2) | ((v == m2) & (v > ninf_v) & (gl < g2)))
    nm2 = jnp.where(b1, m1, jnp.where(b2, v, m2))
    ng2 = jnp.where(b1, g1, jnp.where(b2, gl, g2))
    np2 = jnp.where(b1, p1, jnp.where(b2, pos, p2))
    nm1 = jnp.where(b1, v, m1)
    ng1 = jnp.where(b1, gl, g1)
    np1 = jnp.where(b1, pos, p1)
    return nm1, ng1, np1, nm2, ng2, np2


def _sc_body(cx_hbm, cy_hbm, w_hbm, h_hbm, s_hbm, out_hbm,
             sx1, sy1, sx2, sy2, sarea, sscore, sglin, souts,
             pubrow_v, pub_l, pub_sh):
    sid = lax.axis_index("s")
    base = sid * SHARD
    lane = _lane()
    neg_inf = jnp.float32(-jnp.inf)
    ninf_v = _splat_f(neg_inf)

    pltpu.sync_copy(cx_hbm.at[pl.ds(base, SHARD)], sx1)
    pltpu.sync_copy(w_hbm.at[pl.ds(base, SHARD)], sx2)
    pltpu.sync_copy(cy_hbm.at[pl.ds(base, SHARD)], sy1)
    pltpu.sync_copy(h_hbm.at[pl.ds(base, SHARD)], sy2)
    pltpu.sync_copy(s_hbm.at[pl.ds(base, SHARD)], sscore)

    cnt0 = jnp.minimum(jnp.maximum(N - base, 0), SHARD)

    top0 = (ninf_v, _splat_i(BIG), _splat_i(0),
            ninf_v, _splat_i(BIG), _splat_i(0))

    def init_body(k, carry):
        sl = pl.ds(k * 16, 16)
        va = sx1[sl]
        vb = sx2[sl]
        x1 = va - vb / 2.0
        x2 = va + vb / 2.0
        sx1[sl] = x1
        sx2[sl] = x2
        vc = sy1[sl]
        vd = sy2[sl]
        y1 = vc - vd / 2.0
        y2 = vc + vd / 2.0
        sy1[sl] = y1
        sy2[sl] = y2
        sarea[sl] = (x2 - x1) * (y2 - y1)
        pos = k * 16 + lane
        gl = base + pos
        sglin[sl] = gl
        souts[sl] = jnp.zeros((16,), jnp.float32)
        sc = sscore[sl]
        scm = jnp.where(pos < cnt0, sc, ninf_v)
        return _top2_update(carry, scm, gl, pos, ninf_v)

    top = lax.fori_loop(0, NV, init_body, top0)

    def publish_and_read(top):
        m1, g1, p1, m2, g2, p2 = top
        mA = jnp.max(m1)
        glA = jnp.min(jnp.where((m1 == mA) & (m1 > ninf_v), g1, BIG))
        pA = jnp.min(jnp.where((m1 == mA) & (g1 == glA), p1, BIG))
        is_w = (m1 == mA) & (g1 == glA)
        mBv = jnp.where(is_w, m2, m1)
        gBv = jnp.where(is_w, g2, g1)
        pBv = jnp.where(is_w, p2, p1)
        mB = jnp.max(mBv)
        glB = jnp.min(jnp.where((mBv == mB) & (mBv > ninf_v), gBv, BIG))
        pB = jnp.min(jnp.where((mBv == mB) & (gBv == glB), pBv, BIG))

        pAv = _splat_i(jnp.where(mA > neg_inf, pA, 0))
        pBv_ = _splat_i(jnp.where(mB > neg_inf, pB, 0))
        row = _splat_f(mA)
        row = jnp.where(lane == 1, plsc.bitcast(_splat_i(glA), jnp.float32), row)
        row = jnp.where(lane == 2, plsc.load_gather(sx1, [pAv]), row)
        row = jnp.where(lane == 3, plsc.load_gather(sy1, [pAv]), row)
        row = jnp.where(lane == 4, plsc.load_gather(sx2, [pAv]), row)
        row = jnp.where(lane == 5, plsc.load_gather(sy2, [pAv]), row)
        row = jnp.where(lane == 6, plsc.load_gather(sarea, [pAv]), row)
        row = jnp.where(lane == 7, _splat_f(mB), row)
        row = jnp.where(lane == 8, plsc.bitcast(_splat_i(glB), jnp.float32), row)
        row = jnp.where(lane == 9, plsc.load_gather(sx1, [pBv_]), row)
        row = jnp.where(lane == 10, plsc.load_gather(sy1, [pBv_]), row)
        row = jnp.where(lane == 11, plsc.load_gather(sx2, [pBv_]), row)
        row = jnp.where(lane == 12, plsc.load_gather(sy2, [pBv_]), row)
        row = jnp.where(lane == 13, plsc.load_gather(sarea, [pBv_]), row)
        pubrow_v[...] = row
        pltpu.sync_copy(pubrow_v, pub_sh.at[sid])
        plsc.subcore_barrier()
        pltpu.sync_copy(pub_sh, pub_l)
        plsc.subcore_barrier()

        mv1 = plsc.load_gather(pub_l, [lane, _splat_i(0)])
        gv1 = plsc.bitcast(plsc.load_gather(pub_l, [lane, _splat_i(1)]),
                           jnp.int32)
        mv2 = plsc.load_gather(pub_l, [lane, _splat_i(7)])
        gv2 = plsc.bitcast(plsc.load_gather(pub_l, [lane, _splat_i(8)]),
                           jnp.int32)
        gmax1 = jnp.max(mv1)
        gwin1 = jnp.min(jnp.where(mv1 == gmax1, gv1, BIG))
        t1 = jnp.where(gmax1 > neg_inf, gwin1 // SHARD, 0)
        t1v = _splat_i(t1)
        c1 = [plsc.load_gather(pub_l, [t1v, _splat_i(c)]) for c in range(2, 7)]
        is_t1 = lane == t1
        m2cand = jnp.where(is_t1, mv2, mv1)
        g2cand = jnp.where(is_t1, gv2, gv1)
        gmax2 = jnp.max(m2cand)
        gwin2 = jnp.min(jnp.where(m2cand == gmax2, g2cand, BIG))
        t2 = jnp.where(gmax2 > neg_inf, gwin2 // SHARD, 0)
        colb = jnp.where(t2 == t1, 9, 2)
        t2v = _splat_i(t2)
        c2 = [plsc.load_gather(pub_l, [t2v, _splat_i(colb + c)])
              for c in range(5)]
        ix1 = jnp.maximum(c1[0], c2[0])
        iy1 = jnp.maximum(c1[1], c2[1])
        ix2 = jnp.minimum(c1[2], c2[2])
        iy2 = jnp.minimum(c1[3], c2[3])
        zero = jnp.float32(0.0)
        inter = jnp.maximum(ix2 - ix1, zero) * jnp.maximum(iy2 - iy1, zero)
        iou12 = inter / (c1[4] + c2[4] - inter + 1e-9)
        okv = jnp.logical_not(iou12 > IOU_THRES) & (gmax2 > neg_inf)
        acc = jnp.max(okv.astype(jnp.int32)) > 0
        return (gmax1, gwin1, gmax2, gwin2, acc) + tuple(c1) + tuple(c2)

    r = publish_and_read(top)
    state0 = (jnp.int32(1),) + r + (cnt0,)

    def cond(state):
        return state[1] > neg_inf

    def loop_body(state):
        it = state[0]
        (gmax1, gwin1, gmax2, gwin2, acc,
         ax1, ay1, ax2, ay2, aar, bx1, by1, bx2, by2, bar) = state[1:16]
        count = state[16]
        nv = (count + 15) // 16
        accv = jnp.full((16,), acc, jnp.bool_)
        neg_inf = jnp.float32(-jnp.inf)

        def step(k, carry):
            woff, top = carry
            sl = pl.ds(k * 16, 16)
            x1 = sx1[sl]
            y1 = sy1[sl]
            x2 = sx2[sl]
            y2 = sy2[sl]
            ar = sarea[sl]
            sc = sscore[sl]
            gl = sglin[sl]
            zero = jnp.float32(0.0)
            i1x1 = jnp.maximum(ax1, x1)
            i1y1 = jnp.maximum(ay1, y1)
            i1x2 = jnp.minimum(ax2, x2)
            i1y2 = jnp.minimum(ay2, y2)
            in1 = jnp.maximum(i1x2 - i1x1, zero) * jnp.maximum(i1y2 - i1y1, zero)
            iou1 = in1 / (aar + ar - in1 + 1e-9)
            i2x1 = jnp.maximum(bx1, x1)
            i2y1 = jnp.maximum(by1, y1)
            i2x2 = jnp.minimum(bx2, x2)
            i2y2 = jnp.minimum(by2, y2)
            in2 = jnp.maximum(i2x2 - i2x1, zero) * jnp.maximum(i2y2 - i2y1, zero)
            iou2 = in2 / (bar + ar - in2 + 1e-9)
            inb = (k * 16 + lane) < count
            dead = (iou1 > IOU_THRES) | (gl == gwin1)
            dead = dead | (accv & ((iou2 > IOU_THRES) | (gl == gwin2)))
            alive = inb & jnp.logical_not(dead)
            cs = plsc.cumsum(alive.astype(jnp.int32))
            idx = woff + cs - 1
            plsc.store_scatter(sx1, [idx], x1, mask=alive)
            plsc.store_scatter(sy1, [idx], y1, mask=alive)
            plsc.store_scatter(sx2, [idx], x2, mask=alive)
            plsc.store_scatter(sy2, [idx], y2, mask=alive)
            plsc.store_scatter(sarea, [idx], ar, mask=alive)
            plsc.store_scatter(sscore, [idx], sc, mask=alive)
            plsc.store_scatter(sglin, [idx], gl, mask=alive)
            scm = jnp.where(alive, sc, _splat_f(neg_inf))
            top = _top2_update(top, scm, gl, idx, _splat_f(neg_inf))
            return woff + jnp.max(cs), top

        woff, top = lax.fori_loop(0, nv, step, (jnp.int32(0), top0))

        @pl.when((gwin1 >= base) & (gwin1 < base + SHARD))
        def _():
            plsc.store_scatter(souts, [_splat_i(gwin1 - base)],
                               _splat_f(gmax1), mask=lane == 0)

        @pl.when(acc & (gwin2 >= base) & (gwin2 < base + SHARD))
        def _():
            plsc.store_scatter(souts, [_splat_i(gwin2 - base)],
                               _splat_f(gmax2), mask=lane == 0)

        r = publish_and_read(top)
        return (it + 1,) + r + (woff,)

    lax.while_loop(cond, loop_body, state0)
    pltpu.sync_copy(souts, out_hbm.at[pl.ds(base, SHARD)])


def kernel(boxes, scores):
    n = scores.shape[0]
    pad = NPAD - n
    b = jnp.pad(boxes, ((0, pad), (0, 0)))
    s = jnp.pad(scores, (0, pad))
    mesh = plsc.VectorSubcoreMesh(core_axis_name="c", subcore_axis_name="s",
                                  num_cores=1, num_subcores=NT)
    f32 = jnp.float32
    out = pl.kernel(
        _sc_body,
        out_type=jax.ShapeDtypeStruct((NPAD,), f32),
        mesh=mesh,
        scratch_types=[
            pltpu.VMEM((SHARD,), f32),
            pltpu.VMEM((SHARD,), f32),
            pltpu.VMEM((SHARD,), f32),
            pltpu.VMEM((SHARD,), f32),
            pltpu.VMEM((SHARD,), f32),
            pltpu.VMEM((SHARD,), f32),
            pltpu.VMEM((SHARD,), jnp.int32),
            pltpu.VMEM((SHARD,), f32),
            pltpu.VMEM((16,), f32),
            pltpu.VMEM((16, 16), f32),
            pltpu.VMEM_SHARED((16, 16), f32),
        ],
        compiler_params=pltpu.CompilerParams(needs_layout_passes=False),
    )(b[:, 0], b[:, 1], b[:, 2], b[:, 3], s)
    return out[:n]
